# baseline (speedup 1.0000x reference)
import functools

import jax
import jax.numpy as jnp
from jax.experimental import pallas as pl


def _expert_matmul_kernel(x_ref, w_ref, b_ref, o_ref):
    dk = pl.program_id(1)
    w = w_ref[0]
    B = x_ref.shape[0]
    for bb in range(B):
        acc = jax.lax.dot_general(
            x_ref[bb, 0], w,
            dimension_numbers=(((1,), (1,)), ((), ())),
            preferred_element_type=jnp.float32,
        )

        @pl.when(dk == 0)
        def _init():
            o_ref[bb, 0] = acc + b_ref[0]

        @pl.when(dk != 0)
        def _accum():
            o_ref[bb, 0] += acc


@jax.jit
def kernel(x, W, b):
    B, E, N, D = x.shape
    BK = 512
    Dk = D // BK

    b3 = b.reshape(E, 1, D)
    return pl.pallas_call(
        _expert_matmul_kernel,
        grid=(E, Dk),
        in_specs=[
            pl.BlockSpec((B, 1, N, BK), lambda e, k: (0, e, 0, k)),
            pl.BlockSpec((1, D, BK), lambda e, k: (e, 0, k)),
            pl.BlockSpec((1, 1, D), lambda e, k: (e, 0, 0)),
        ],
        out_specs=pl.BlockSpec((B, 1, N, D), lambda e, k: (0, e, 0, 0)),
        out_shape=jax.ShapeDtypeStruct((B, E, N, D), x.dtype),
    )(x, W, b3)


# speedup vs baseline: 1.1971x; 1.1971x over previous
import functools

import jax
import jax.numpy as jnp
from jax.experimental import pallas as pl


def _expert_matmul_kernel(x_ref, w_ref, b_ref, o_ref):
    dk = pl.program_id(1)
    w = w_ref[0]
    B = x_ref.shape[0]
    for bb in range(B):
        acc = jax.lax.dot_general(
            x_ref[bb, 0], w,
            dimension_numbers=(((1,), (1,)), ((), ())),
            preferred_element_type=jnp.float32,
        )

        @pl.when(dk == 0)
        def _init():
            o_ref[bb, 0] = acc + b_ref[0]

        @pl.when(dk != 0)
        def _accum():
            o_ref[bb, 0] += acc


@jax.jit
def kernel(x, W, b):
    B, E, N, D = x.shape
    BK = 1024
    Dk = D // BK

    b3 = b.reshape(E, 1, D)
    return pl.pallas_call(
        _expert_matmul_kernel,
        grid=(E, Dk),
        in_specs=[
            pl.BlockSpec((B, 1, N, BK), lambda e, k: (0, e, 0, k)),
            pl.BlockSpec((1, D, BK), lambda e, k: (e, 0, k)),
            pl.BlockSpec((1, 1, D), lambda e, k: (e, 0, 0)),
        ],
        out_specs=pl.BlockSpec((B, 1, N, D), lambda e, k: (0, e, 0, 0)),
        out_shape=jax.ShapeDtypeStruct((B, E, N, D), x.dtype),
    )(x, W, b3)
